# baseline (speedup 1.0000x reference)
import functools
import jax
import jax.numpy as jnp
from jax import lax
from jax.experimental import pallas as pl
from jax.experimental.pallas import tpu as pltpu
from jax.experimental.pallas import tpu_sc as plsc

NSAMPLE = 16
HID = 64
N = 4096
B = 2
QBLK = 512
EPS = 1e-5

_PAR1 = pltpu.CompilerParams(dimension_semantics=("parallel",))


def _leaky(x):
    return jnp.where(x >= 0, x, 0.1 * x)


def _knn_body(q_ref, rt_ref, o_ref):
    q = q_ref[...]
    rt = rt_ref[...]
    n = rt.shape[1]
    qn = jnp.sum(q * q, axis=1, keepdims=True)
    rn = jnp.sum(rt * rt, axis=0, keepdims=True)
    d = qn + rn - 2.0 * jnp.dot(q, rt, preferred_element_type=jnp.float32)
    iota = lax.broadcasted_iota(jnp.int32, d.shape, 1)
    cols = []
    for _ in range(NSAMPLE):
        m = jnp.min(d, axis=1, keepdims=True)
        mi = jnp.where(d <= m, iota, jnp.int32(n))
        j = jnp.min(mi, axis=1, keepdims=True)
        cols.append(j)
        d = jnp.where(iota == j, jnp.float32(jnp.inf), d)
    o_ref[...] = jnp.concatenate(cols, axis=1)


def _knn(queries, refs):
    idx = pl.pallas_call(
        _knn_body,
        grid=(N // QBLK,),
        in_specs=[pl.BlockSpec((QBLK, 3), lambda i: (i, 0)),
                  pl.BlockSpec((3, N), lambda i: (0, 0))],
        out_specs=pl.BlockSpec((QBLK, NSAMPLE), lambda i: (i, 0)),
        out_shape=jax.ShapeDtypeStruct((N, NSAMPLE), jnp.int32),
        compiler_params=_PAR1,
    )(queries, refs.T)
    return idx.reshape(-1)


def _sc_gather(table, idx):
    M = idx.shape[0]
    D = table.shape[1]
    NW = 32
    per_w = M // NW
    ch = min(per_w, 512)
    mesh = plsc.VectorSubcoreMesh(core_axis_name="c", subcore_axis_name="s")

    @functools.partial(
        pl.kernel, mesh=mesh,
        out_type=jax.ShapeDtypeStruct((M, D), jnp.float32),
        scratch_types=[
            pltpu.VMEM((ch,), jnp.int32),
            pltpu.VMEM((ch, D), jnp.float32),
            pltpu.SemaphoreType.DMA,
        ],
    )
    def k(table_hbm, idx_hbm, out_hbm, idx_v, rows_v, sem):
        wid = lax.axis_index("s") * 2 + lax.axis_index("c")
        base = wid * per_w

        @pl.loop(0, per_w, step=ch)
        def _(off):
            pltpu.sync_copy(idx_hbm.at[pl.ds(base + off, ch)], idx_v)
            pltpu.async_copy(table_hbm.at[idx_v], rows_v, sem).wait()
            pltpu.sync_copy(rows_v, out_hbm.at[pl.ds(base + off, ch)])

    return k(table, idx)


FBLK = 1024
FNB = N // FBLK
_FCNT = float(N * NSAMPLE * 16)


def _y1_block(r1, px, b1):
    pb = jnp.broadcast_to(px[:, None, :], (FBLK, NSAMPLE, 64)).reshape(
        FBLK * NSAMPLE, 64)
    return r1[:, 0:64] - pb + b1


def _gstats(y):
    parts = []
    for g in range(4):
        blkg = y[:, 16 * g:16 * (g + 1)]
        parts.append(jnp.sum(blkg).reshape(1, 1, 1))
    for g in range(4):
        blkg = y[:, 16 * g:16 * (g + 1)]
        parts.append(jnp.sum(blkg * blkg).reshape(1, 1, 1))
    return jnp.concatenate(parts, axis=2)


def _gn_apply(y, stats, gamma, beta):
    s = jnp.sum(stats.reshape(FNB, 8), axis=0)
    outs = []
    for g in range(4):
        m = s[g] / _FCNT
        v = s[4 + g] / _FCNT - m * m
        blkg = y[:, 16 * g:16 * (g + 1)]
        outs.append((blkg - m) * lax.rsqrt(v + EPS))
    yn = jnp.concatenate(outs, axis=1) * gamma + beta
    return _leaky(yn)


def _flow_s1_body(r1_ref, px_ref, b1_ref, st_ref):
    st_ref[...] = _gstats(_y1_block(r1_ref[...], px_ref[...], b1_ref[...]))


def _flow_s2_body(r1_ref, px_ref, b1_ref, st1_ref, g1_ref, be1_ref,
                  w2_ref, b2_ref, y2_ref, st2_ref):
    y1 = _y1_block(r1_ref[...], px_ref[...], b1_ref[...])
    h = _gn_apply(y1, st1_ref[...], g1_ref[...], be1_ref[...])
    y2 = jnp.dot(h, w2_ref[...].T, preferred_element_type=jnp.float32) \
        + b2_ref[...]
    y2_ref[...] = y2
    st2_ref[...] = _gstats(y2)


def _flow_s3_body(y2_ref, st2_ref, g2_ref, be2_ref, c_ref, s_ref, p_ref,
                  wz_ref, wr_ref, o_ref, azr_ref, px_ref):
    h2 = _gn_apply(y2_ref[...], st2_ref[...], g2_ref[...], be2_ref[...])
    ff = jnp.max(h2.reshape(FBLK, NSAMPLE, 64), axis=1)
    o_ref[...] = ff
    hs = jnp.concatenate([c_ref[...], ff, s_ref[...]], axis=1)
    p0 = p_ref[...]
    wz = wz_ref[...]
    wr = wr_ref[...]
    az = (jnp.dot(hs, wz[:, 0:192].T, preferred_element_type=jnp.float32)
          + jnp.dot(p0, wz[:, 192:195].T, preferred_element_type=jnp.float32))
    ar = (jnp.dot(hs, wr[:, 0:192].T, preferred_element_type=jnp.float32)
          + jnp.dot(p0, wr[:, 192:195].T, preferred_element_type=jnp.float32))
    azr_ref[...] = jnp.concatenate([az, ar], axis=1)
    pxz = jnp.dot(p0, wz[:, 192:195].T, preferred_element_type=jnp.float32)
    pxr = jnp.dot(p0, wr[:, 192:195].T, preferred_element_type=jnp.float32)
    px_ref[...] = jnp.concatenate([pxz, pxr], axis=1)


def _flow_stage(r1, px1, b1, g1, be1, w2, b2, g2, be2, c, s, p, wz, wr):
    grid = (FNB,)
    rblk = pl.BlockSpec((FBLK * NSAMPLE, 128), lambda i: (i, 0))
    yblk = pl.BlockSpec((FBLK * NSAMPLE, 64), lambda i: (i, 0))
    pblk = pl.BlockSpec((FBLK, 64), lambda i: (i, 0))
    vec = pl.BlockSpec((64,), lambda i: (0,))
    st_out = pl.BlockSpec((1, 1, 8), lambda i: (i, 0, 0))
    st_in = pl.BlockSpec((FNB, 1, 8), lambda i: (0, 0, 0))
    st_shape = jax.ShapeDtypeStruct((FNB, 1, 8), jnp.float32)

    st1 = pl.pallas_call(
        _flow_s1_body, grid=grid,
        in_specs=[rblk, pblk, vec],
        out_specs=st_out, out_shape=st_shape,
        compiler_params=_PAR1,
    )(r1, px1, b1)

    y2, st2 = pl.pallas_call(
        _flow_s2_body, grid=grid,
        in_specs=[rblk, pblk, vec, st_in, vec, vec,
                  pl.BlockSpec((64, 64), lambda i: (0, 0)), vec],
        out_specs=[yblk, st_out],
        out_shape=[jax.ShapeDtypeStruct((N * NSAMPLE, 64), jnp.float32),
                   st_shape],
        compiler_params=_PAR1,
    )(r1, px1, b1, st1, g1, be1, w2, b2)

    return pl.pallas_call(
        _flow_s3_body, grid=grid,
        in_specs=[yblk, st_in, vec, vec, pblk, pblk,
                  pl.BlockSpec((FBLK, 3), lambda i: (i, 0)),
                  pl.BlockSpec((64, 195), lambda i: (0, 0)),
                  pl.BlockSpec((64, 195), lambda i: (0, 0))],
        out_specs=[pblk, pl.BlockSpec((FBLK, 128), lambda i: (i, 0)),
                   pl.BlockSpec((FBLK, 128), lambda i: (i, 0))],
        out_shape=[jax.ShapeDtypeStruct((N, 64), jnp.float32),
                   jax.ShapeDtypeStruct((N, 128), jnp.float32),
                   jax.ShapeDtypeStruct((N, 128), jnp.float32)],
        compiler_params=_PAR1,
    )(y2, st2, g2, be2, c, s, p, wz, wr)


def _a1_body(fl_ref, p0_ref, w1_ref, a1_ref, px_ref):
    w1 = w1_ref[...]
    a1 = (jnp.dot(fl_ref[...], w1[:, 0:3].T, preferred_element_type=jnp.float32)
          + jnp.dot(p0_ref[...], w1[:, 3:6].T, preferred_element_type=jnp.float32))
    a1_ref[...] = jnp.concatenate(
        [a1, jnp.zeros((a1.shape[0], 64), jnp.float32)], axis=1)
    px_ref[...] = jnp.dot(p0_ref[...], w1[:, 3:6].T,
                          preferred_element_type=jnp.float32)


def _a1_pre(fl0, p0, w1):
    blk = lambda d: pl.BlockSpec((N, d), lambda: (0, 0))
    return pl.pallas_call(
        _a1_body,
        in_specs=[blk(3), blk(3), pl.BlockSpec((64, 6), lambda: (0, 0))],
        out_specs=[blk(128), blk(64)],
        out_shape=[jax.ShapeDtypeStruct((N, 128), jnp.float32),
                   jax.ShapeDtypeStruct((N, 64), jnp.float32)],
    )(fl0, p0, w1)


ZBLK = 1024
ZNB = N // ZBLK


def _gru_mid_body(g_ref, px_ref, c_ref, f_ref, s_ref, p_ref,
                  wq_ref, bz_ref, br_ref, sq_ref, z_ref, pxq_ref):
    mzr = jnp.max(g_ref[...].reshape(ZBLK, NSAMPLE, 128), axis=1)
    px = px_ref[...]
    z = jax.nn.sigmoid(mzr[:, 0:64] - px[:, 0:64] + bz_ref[...])
    r = jax.nn.sigmoid(mzr[:, 64:128] - px[:, 64:128] + br_ref[...])
    st = s_ref[...]
    rs = r * st
    feat = jnp.concatenate([c_ref[...], f_ref[...]], axis=1)
    wq = wq_ref[...]
    p0 = p_ref[...]
    sq = (jnp.dot(feat, wq[:, 0:128].T, preferred_element_type=jnp.float32)
          + jnp.dot(rs, wq[:, 128:192].T, preferred_element_type=jnp.float32)
          + jnp.dot(p0, wq[:, 192:195].T, preferred_element_type=jnp.float32))
    sq_ref[...] = jnp.concatenate(
        [sq, jnp.zeros((sq.shape[0], 64), jnp.float32)], axis=1)
    z_ref[...] = z
    pxq_ref[...] = jnp.dot(p0, wq[:, 192:195].T,
                           preferred_element_type=jnp.float32)


def _gru_mid(gzr, px, c, f, s, p, wq, bz, br):
    blk = lambda d: pl.BlockSpec((ZBLK, d), lambda g: (g, 0))
    vec = pl.BlockSpec((64,), lambda g: (0,))
    return pl.pallas_call(
        _gru_mid_body,
        grid=(ZNB,),
        in_specs=[pl.BlockSpec((ZBLK * NSAMPLE, 128), lambda g: (g, 0)),
                  blk(128), blk(64), blk(64), blk(64), blk(3),
                  pl.BlockSpec((64, 195), lambda g: (0, 0)), vec, vec],
        out_specs=[blk(128), blk(64), blk(64)],
        out_shape=[jax.ShapeDtypeStruct((N, 128), jnp.float32),
                   jax.ShapeDtypeStruct((N, 64), jnp.float32),
                   jax.ShapeDtypeStruct((N, 64), jnp.float32)],
        compiler_params=_PAR1,
    )(gzr, px, c, f, s, p, wq, bz, br)


def _gru_fin_body(g_ref, pxq_ref, z_ref, s_ref, p0_ref, p1_ref,
                  f0_ref, f1_ref, bq_ref, wa_ref, ba_ref, gv_ref, qq_ref):
    mq = jnp.max(g_ref[...][:, 0:64].reshape(ZBLK, NSAMPLE, 64), axis=1)
    q = jnp.tanh(mq - pxq_ref[...] + bq_ref[...])
    z = z_ref[...]
    ns = (1.0 - z) * s_ref[...] + z * q
    wa = wa_ref[...]
    g_t = (jnp.dot(f0_ref[...], wa[:, 0:64].T, preferred_element_type=jnp.float32)
           + jnp.dot(p0_ref[...], wa[:, 128:131].T, preferred_element_type=jnp.float32))
    qq = (jnp.dot(f1_ref[...], wa[:, 64:128].T, preferred_element_type=jnp.float32)
          - jnp.dot(p1_ref[...], wa[:, 128:131].T, preferred_element_type=jnp.float32)
          + ba_ref[...])
    gv_ref[...] = jnp.concatenate([g_t, ns], axis=1)
    qq_ref[...] = qq


def _gru_fin(gq, pxq, z, s, p0, p1, f0, f1, bq, wa, ba):
    blk = lambda d: pl.BlockSpec((ZBLK, d), lambda g: (g, 0))
    vec = pl.BlockSpec((64,), lambda g: (0,))
    return pl.pallas_call(
        _gru_fin_body,
        grid=(ZNB,),
        in_specs=[pl.BlockSpec((ZBLK * NSAMPLE, 128), lambda g: (g, 0)),
                  blk(64), blk(64), blk(64), blk(3), blk(3), blk(64), blk(64),
                  vec, pl.BlockSpec((64, 131), lambda g: (0, 0)), vec],
        out_specs=[blk(128), blk(64)],
        out_shape=[jax.ShapeDtypeStruct((N, 128), jnp.float32),
                   jax.ShapeDtypeStruct((N, 64), jnp.float32)],
        compiler_params=_PAR1,
    )(gq, pxq, z, s, p0, p1, f0, f1, bq, wa, ba)


ABLK = 512


def _align_body(r2_ref, qq_ref, w2_ref, b2_ref, o_ref):
    r2 = r2_ref[...]
    qq = qq_ref[...]
    qb = jnp.broadcast_to(qq[:, None, :], (ABLK, NSAMPLE, 64)).reshape(
        ABLK * NSAMPLE, 64)
    h = _leaky(r2[:, 0:64] + qb)
    y = jnp.dot(h, w2_ref[...].T, preferred_element_type=jnp.float32) + b2_ref[...]
    y3 = y.reshape(ABLK, NSAMPLE, 64)
    m = jnp.max(y3, axis=1, keepdims=True)
    e = jnp.exp(y3 - m)
    w = e / jnp.sum(e, axis=1, keepdims=True)
    v3 = r2[:, 64:128].reshape(ABLK, NSAMPLE, 64)
    o_ref[...] = jnp.sum(w * v3, axis=1)


def _align(r2, qq, w2, b2):
    return pl.pallas_call(
        _align_body,
        grid=(N // ABLK,),
        in_specs=[pl.BlockSpec((ABLK * NSAMPLE, 128), lambda g: (g, 0)),
                  pl.BlockSpec((ABLK, 64), lambda g: (g, 0)),
                  pl.BlockSpec((64, 64), lambda g: (0, 0)),
                  pl.BlockSpec((64,), lambda g: (0,))],
        out_specs=pl.BlockSpec((ABLK, 64), lambda g: (g, 0)),
        out_shape=jax.ShapeDtypeStruct((N, 64), jnp.float32),
        compiler_params=_PAR1,
    )(r2, qq, w2, b2)


def kernel(xyz0, xyz1, state, corr0, feat0, feat1, flow0,
           flow_w1, flow_b1, flow_g1, flow_be1,
           flow_w2, flow_b2, flow_g2, flow_be2,
           convz_w, convz_b, convr_w, convr_b, convq_w, convq_b,
           interp_w1, interp_b1, interp_w2, interp_b2):
    t = lambda x: x.transpose(0, 2, 1)
    p0a = t(xyz0)
    p1a = t(xyz1)
    sta = t(state)
    c0a = t(corr0)
    f0a = t(feat0)
    f1a = t(feat1)
    fla = t(flow0)

    outs = []
    for b in range(B):
        p0, p1, st, c0 = p0a[b], p1a[b], sta[b], c0a[b]
        f0, f1, fl = f0a[b], f1a[b], fla[b]

        idxg0 = _knn(p0, p0)
        idxg1 = _knn(p1, p0)

        a1, px1 = _a1_pre(fl, p0, flow_w1)
        r1 = _sc_gather(a1, idxg0)
        ff0, azr, px = _flow_stage(
            r1, px1, flow_b1, flow_g1, flow_be1,
            flow_w2, flow_b2, flow_g2, flow_be2,
            c0, st, p0, convz_w, convr_w)

        gzr = _sc_gather(azr, idxg0)
        sq, z, pxq = _gru_mid(gzr, px, c0, ff0, st, p0, convq_w,
                              convz_b, convr_b)
        gq = _sc_gather(sq, idxg0)
        gv, qq = _gru_fin(gq, pxq, z, st, p0, p1, f0, f1,
                          convq_b, interp_w1, interp_b1)

        r2 = _sc_gather(gv, idxg1)
        outs.append(_align(r2, qq, interp_w2, interp_b2))

    return jnp.stack(outs).transpose(0, 2, 1)


# speedup vs baseline: 1.0020x; 1.0020x over previous
import functools
import jax
import jax.numpy as jnp
from jax import lax
from jax.experimental import pallas as pl
from jax.experimental.pallas import tpu as pltpu
from jax.experimental.pallas import tpu_sc as plsc

NSAMPLE = 16
HID = 64
N = 4096
B = 2
QBLK = 256
EPS = 1e-5

_PAR1 = pltpu.CompilerParams(dimension_semantics=("parallel",))


def _leaky(x):
    return jnp.where(x >= 0, x, 0.1 * x)


def _knn_body(q_ref, rt_ref, o_ref):
    q = q_ref[...]
    rt = rt_ref[...]
    n = rt.shape[1]
    qn = jnp.sum(q * q, axis=1, keepdims=True)
    rn = jnp.sum(rt * rt, axis=0, keepdims=True)
    d = qn + rn - 2.0 * jnp.dot(q, rt, preferred_element_type=jnp.float32)
    iota = lax.broadcasted_iota(jnp.int32, d.shape, 1)
    cols = []
    for _ in range(NSAMPLE):
        m = jnp.min(d, axis=1, keepdims=True)
        mi = jnp.where(d <= m, iota, jnp.int32(n))
        j = jnp.min(mi, axis=1, keepdims=True)
        cols.append(j)
        d = jnp.where(iota == j, jnp.float32(jnp.inf), d)
    o_ref[...] = jnp.concatenate(cols, axis=1)


def _knn(queries, refs):
    idx = pl.pallas_call(
        _knn_body,
        grid=(N // QBLK,),
        in_specs=[pl.BlockSpec((QBLK, 3), lambda i: (i, 0)),
                  pl.BlockSpec((3, N), lambda i: (0, 0))],
        out_specs=pl.BlockSpec((QBLK, NSAMPLE), lambda i: (i, 0)),
        out_shape=jax.ShapeDtypeStruct((N, NSAMPLE), jnp.int32),
        compiler_params=_PAR1,
    )(queries, refs.T)
    return idx.reshape(-1)


def _sc_gather(table, idx):
    M = idx.shape[0]
    D = table.shape[1]
    NW = 32
    per_w = M // NW
    ch = min(per_w, 256)
    nch = per_w // ch
    mesh = plsc.VectorSubcoreMesh(core_axis_name="c", subcore_axis_name="s")

    @functools.partial(
        pl.kernel, mesh=mesh,
        out_type=jax.ShapeDtypeStruct((M, D), jnp.float32),
        scratch_types=[
            pltpu.VMEM((ch,), jnp.int32),
            pltpu.VMEM((ch,), jnp.int32),
            pltpu.VMEM((ch, D), jnp.float32),
            pltpu.VMEM((ch, D), jnp.float32),
            pltpu.SemaphoreType.DMA,
            pltpu.SemaphoreType.DMA,
            pltpu.SemaphoreType.DMA,
        ],
    )
    def k(table_hbm, idx_hbm, out_hbm, i0, i1, r0, r1, gsem, osem0, osem1):
        wid = lax.axis_index("s") * 2 + lax.axis_index("c")
        base = wid * per_w
        ibufs = (i0, i1)
        rbufs = (r0, r1)
        del osem0, osem1

        pltpu.sync_copy(idx_hbm.at[pl.ds(base, ch)], i0)
        pltpu.async_copy(table_hbm.at[i0], r0, gsem)
        for j in range(nch):
            s = j % 2
            if j + 1 < nch:
                pltpu.sync_copy(
                    idx_hbm.at[pl.ds(base + (j + 1) * ch, ch)], ibufs[s ^ 1])
            pltpu.make_async_copy(table_hbm.at[ibufs[s]], rbufs[s],
                                  gsem).wait()
            if j + 1 < nch:
                pltpu.async_copy(table_hbm.at[ibufs[s ^ 1]], rbufs[s ^ 1],
                                 gsem)
            pltpu.sync_copy(rbufs[s], out_hbm.at[pl.ds(base + j * ch, ch)])

    return k(table, idx)


FBLK = 1024
FNB = N // FBLK
_FCNT = float(N * NSAMPLE * 16)


def _y1_block(r1, px, b1):
    pb = jnp.broadcast_to(px[:, None, :], (FBLK, NSAMPLE, 64)).reshape(
        FBLK * NSAMPLE, 64)
    return r1[:, 0:64] - pb + b1


def _gstats(y):
    parts = []
    for g in range(4):
        blkg = y[:, 16 * g:16 * (g + 1)]
        parts.append(jnp.sum(blkg).reshape(1, 1, 1))
    for g in range(4):
        blkg = y[:, 16 * g:16 * (g + 1)]
        parts.append(jnp.sum(blkg * blkg).reshape(1, 1, 1))
    return jnp.concatenate(parts, axis=2)


def _gn_apply(y, stats, gamma, beta):
    s = jnp.sum(stats.reshape(FNB, 8), axis=0)
    outs = []
    for g in range(4):
        m = s[g] / _FCNT
        v = s[4 + g] / _FCNT - m * m
        blkg = y[:, 16 * g:16 * (g + 1)]
        outs.append((blkg - m) * lax.rsqrt(v + EPS))
    yn = jnp.concatenate(outs, axis=1) * gamma + beta
    return _leaky(yn)


def _flow_s1_body(r1_ref, px_ref, b1_ref, st_ref):
    st_ref[...] = _gstats(_y1_block(r1_ref[...], px_ref[...], b1_ref[...]))


def _flow_s2_body(r1_ref, px_ref, b1_ref, st1_ref, g1_ref, be1_ref,
                  w2_ref, b2_ref, y2_ref, st2_ref):
    y1 = _y1_block(r1_ref[...], px_ref[...], b1_ref[...])
    h = _gn_apply(y1, st1_ref[...], g1_ref[...], be1_ref[...])
    y2 = jnp.dot(h, w2_ref[...].T, preferred_element_type=jnp.float32) \
        + b2_ref[...]
    y2_ref[...] = y2
    st2_ref[...] = _gstats(y2)


def _flow_s3_body(y2_ref, st2_ref, g2_ref, be2_ref, c_ref, s_ref, p_ref,
                  wz_ref, wr_ref, o_ref, azr_ref, px_ref):
    h2 = _gn_apply(y2_ref[...], st2_ref[...], g2_ref[...], be2_ref[...])
    ff = jnp.max(h2.reshape(FBLK, NSAMPLE, 64), axis=1)
    o_ref[...] = ff
    hs = jnp.concatenate([c_ref[...], ff, s_ref[...]], axis=1)
    p0 = p_ref[...]
    wz = wz_ref[...]
    wr = wr_ref[...]
    az = (jnp.dot(hs, wz[:, 0:192].T, preferred_element_type=jnp.float32)
          + jnp.dot(p0, wz[:, 192:195].T, preferred_element_type=jnp.float32))
    ar = (jnp.dot(hs, wr[:, 0:192].T, preferred_element_type=jnp.float32)
          + jnp.dot(p0, wr[:, 192:195].T, preferred_element_type=jnp.float32))
    azr_ref[...] = jnp.concatenate([az, ar], axis=1)
    pxz = jnp.dot(p0, wz[:, 192:195].T, preferred_element_type=jnp.float32)
    pxr = jnp.dot(p0, wr[:, 192:195].T, preferred_element_type=jnp.float32)
    px_ref[...] = jnp.concatenate([pxz, pxr], axis=1)


def _flow_stage(r1, px1, b1, g1, be1, w2, b2, g2, be2, c, s, p, wz, wr):
    grid = (FNB,)
    rblk = pl.BlockSpec((FBLK * NSAMPLE, 128), lambda i: (i, 0))
    yblk = pl.BlockSpec((FBLK * NSAMPLE, 64), lambda i: (i, 0))
    pblk = pl.BlockSpec((FBLK, 64), lambda i: (i, 0))
    vec = pl.BlockSpec((64,), lambda i: (0,))
    st_out = pl.BlockSpec((1, 1, 8), lambda i: (i, 0, 0))
    st_in = pl.BlockSpec((FNB, 1, 8), lambda i: (0, 0, 0))
    st_shape = jax.ShapeDtypeStruct((FNB, 1, 8), jnp.float32)

    st1 = pl.pallas_call(
        _flow_s1_body, grid=grid,
        in_specs=[rblk, pblk, vec],
        out_specs=st_out, out_shape=st_shape,
        compiler_params=_PAR1,
    )(r1, px1, b1)

    y2, st2 = pl.pallas_call(
        _flow_s2_body, grid=grid,
        in_specs=[rblk, pblk, vec, st_in, vec, vec,
                  pl.BlockSpec((64, 64), lambda i: (0, 0)), vec],
        out_specs=[yblk, st_out],
        out_shape=[jax.ShapeDtypeStruct((N * NSAMPLE, 64), jnp.float32),
                   st_shape],
        compiler_params=_PAR1,
    )(r1, px1, b1, st1, g1, be1, w2, b2)

    return pl.pallas_call(
        _flow_s3_body, grid=grid,
        in_specs=[yblk, st_in, vec, vec, pblk, pblk,
                  pl.BlockSpec((FBLK, 3), lambda i: (i, 0)),
                  pl.BlockSpec((64, 195), lambda i: (0, 0)),
                  pl.BlockSpec((64, 195), lambda i: (0, 0))],
        out_specs=[pblk, pl.BlockSpec((FBLK, 128), lambda i: (i, 0)),
                   pl.BlockSpec((FBLK, 128), lambda i: (i, 0))],
        out_shape=[jax.ShapeDtypeStruct((N, 64), jnp.float32),
                   jax.ShapeDtypeStruct((N, 128), jnp.float32),
                   jax.ShapeDtypeStruct((N, 128), jnp.float32)],
        compiler_params=_PAR1,
    )(y2, st2, g2, be2, c, s, p, wz, wr)


def _a1_body(fl_ref, p0_ref, w1_ref, a1_ref, px_ref):
    w1 = w1_ref[...]
    a1 = (jnp.dot(fl_ref[...], w1[:, 0:3].T, preferred_element_type=jnp.float32)
          + jnp.dot(p0_ref[...], w1[:, 3:6].T, preferred_element_type=jnp.float32))
    a1_ref[...] = jnp.concatenate(
        [a1, jnp.zeros((a1.shape[0], 64), jnp.float32)], axis=1)
    px_ref[...] = jnp.dot(p0_ref[...], w1[:, 3:6].T,
                          preferred_element_type=jnp.float32)


def _a1_pre(fl0, p0, w1):
    blk = lambda d: pl.BlockSpec((N, d), lambda: (0, 0))
    return pl.pallas_call(
        _a1_body,
        in_specs=[blk(3), blk(3), pl.BlockSpec((64, 6), lambda: (0, 0))],
        out_specs=[blk(128), blk(64)],
        out_shape=[jax.ShapeDtypeStruct((N, 128), jnp.float32),
                   jax.ShapeDtypeStruct((N, 64), jnp.float32)],
    )(fl0, p0, w1)


ZBLK = 1024
ZNB = N // ZBLK


def _gru_mid_body(g_ref, px_ref, c_ref, f_ref, s_ref, p_ref,
                  wq_ref, bz_ref, br_ref, sq_ref, z_ref, pxq_ref):
    mzr = jnp.max(g_ref[...].reshape(ZBLK, NSAMPLE, 128), axis=1)
    px = px_ref[...]
    z = jax.nn.sigmoid(mzr[:, 0:64] - px[:, 0:64] + bz_ref[...])
    r = jax.nn.sigmoid(mzr[:, 64:128] - px[:, 64:128] + br_ref[...])
    st = s_ref[...]
    rs = r * st
    feat = jnp.concatenate([c_ref[...], f_ref[...]], axis=1)
    wq = wq_ref[...]
    p0 = p_ref[...]
    sq = (jnp.dot(feat, wq[:, 0:128].T, preferred_element_type=jnp.float32)
          + jnp.dot(rs, wq[:, 128:192].T, preferred_element_type=jnp.float32)
          + jnp.dot(p0, wq[:, 192:195].T, preferred_element_type=jnp.float32))
    sq_ref[...] = jnp.concatenate(
        [sq, jnp.zeros((sq.shape[0], 64), jnp.float32)], axis=1)
    z_ref[...] = z
    pxq_ref[...] = jnp.dot(p0, wq[:, 192:195].T,
                           preferred_element_type=jnp.float32)


def _gru_mid(gzr, px, c, f, s, p, wq, bz, br):
    blk = lambda d: pl.BlockSpec((ZBLK, d), lambda g: (g, 0))
    vec = pl.BlockSpec((64,), lambda g: (0,))
    return pl.pallas_call(
        _gru_mid_body,
        grid=(ZNB,),
        in_specs=[pl.BlockSpec((ZBLK * NSAMPLE, 128), lambda g: (g, 0)),
                  blk(128), blk(64), blk(64), blk(64), blk(3),
                  pl.BlockSpec((64, 195), lambda g: (0, 0)), vec, vec],
        out_specs=[blk(128), blk(64), blk(64)],
        out_shape=[jax.ShapeDtypeStruct((N, 128), jnp.float32),
                   jax.ShapeDtypeStruct((N, 64), jnp.float32),
                   jax.ShapeDtypeStruct((N, 64), jnp.float32)],
        compiler_params=_PAR1,
    )(gzr, px, c, f, s, p, wq, bz, br)


def _gru_fin_body(g_ref, pxq_ref, z_ref, s_ref, p0_ref, p1_ref,
                  f0_ref, f1_ref, bq_ref, wa_ref, ba_ref, gv_ref, qq_ref):
    mq = jnp.max(g_ref[...][:, 0:64].reshape(ZBLK, NSAMPLE, 64), axis=1)
    q = jnp.tanh(mq - pxq_ref[...] + bq_ref[...])
    z = z_ref[...]
    ns = (1.0 - z) * s_ref[...] + z * q
    wa = wa_ref[...]
    g_t = (jnp.dot(f0_ref[...], wa[:, 0:64].T, preferred_element_type=jnp.float32)
           + jnp.dot(p0_ref[...], wa[:, 128:131].T, preferred_element_type=jnp.float32))
    qq = (jnp.dot(f1_ref[...], wa[:, 64:128].T, preferred_element_type=jnp.float32)
          - jnp.dot(p1_ref[...], wa[:, 128:131].T, preferred_element_type=jnp.float32)
          + ba_ref[...])
    gv_ref[...] = jnp.concatenate([g_t, ns], axis=1)
    qq_ref[...] = qq


def _gru_fin(gq, pxq, z, s, p0, p1, f0, f1, bq, wa, ba):
    blk = lambda d: pl.BlockSpec((ZBLK, d), lambda g: (g, 0))
    vec = pl.BlockSpec((64,), lambda g: (0,))
    return pl.pallas_call(
        _gru_fin_body,
        grid=(ZNB,),
        in_specs=[pl.BlockSpec((ZBLK * NSAMPLE, 128), lambda g: (g, 0)),
                  blk(64), blk(64), blk(64), blk(3), blk(3), blk(64), blk(64),
                  vec, pl.BlockSpec((64, 131), lambda g: (0, 0)), vec],
        out_specs=[blk(128), blk(64)],
        out_shape=[jax.ShapeDtypeStruct((N, 128), jnp.float32),
                   jax.ShapeDtypeStruct((N, 64), jnp.float32)],
        compiler_params=_PAR1,
    )(gq, pxq, z, s, p0, p1, f0, f1, bq, wa, ba)


ABLK = 512


def _align_body(r2_ref, qq_ref, w2_ref, b2_ref, o_ref):
    r2 = r2_ref[...]
    qq = qq_ref[...]
    qb = jnp.broadcast_to(qq[:, None, :], (ABLK, NSAMPLE, 64)).reshape(
        ABLK * NSAMPLE, 64)
    h = _leaky(r2[:, 0:64] + qb)
    y = jnp.dot(h, w2_ref[...].T, preferred_element_type=jnp.float32) + b2_ref[...]
    y3 = y.reshape(ABLK, NSAMPLE, 64)
    m = jnp.max(y3, axis=1, keepdims=True)
    e = jnp.exp(y3 - m)
    w = e / jnp.sum(e, axis=1, keepdims=True)
    v3 = r2[:, 64:128].reshape(ABLK, NSAMPLE, 64)
    o_ref[...] = jnp.sum(w * v3, axis=1)


def _align(r2, qq, w2, b2):
    return pl.pallas_call(
        _align_body,
        grid=(N // ABLK,),
        in_specs=[pl.BlockSpec((ABLK * NSAMPLE, 128), lambda g: (g, 0)),
                  pl.BlockSpec((ABLK, 64), lambda g: (g, 0)),
                  pl.BlockSpec((64, 64), lambda g: (0, 0)),
                  pl.BlockSpec((64,), lambda g: (0,))],
        out_specs=pl.BlockSpec((ABLK, 64), lambda g: (g, 0)),
        out_shape=jax.ShapeDtypeStruct((N, 64), jnp.float32),
        compiler_params=_PAR1,
    )(r2, qq, w2, b2)


def kernel(xyz0, xyz1, state, corr0, feat0, feat1, flow0,
           flow_w1, flow_b1, flow_g1, flow_be1,
           flow_w2, flow_b2, flow_g2, flow_be2,
           convz_w, convz_b, convr_w, convr_b, convq_w, convq_b,
           interp_w1, interp_b1, interp_w2, interp_b2):
    t = lambda x: x.transpose(0, 2, 1)
    p0a = t(xyz0)
    p1a = t(xyz1)
    sta = t(state)
    c0a = t(corr0)
    f0a = t(feat0)
    f1a = t(feat1)
    fla = t(flow0)

    outs = []
    for b in range(B):
        p0, p1, st, c0 = p0a[b], p1a[b], sta[b], c0a[b]
        f0, f1, fl = f0a[b], f1a[b], fla[b]

        idxg0 = _knn(p0, p0)
        idxg1 = _knn(p1, p0)

        a1, px1 = _a1_pre(fl, p0, flow_w1)
        r1 = _sc_gather(a1, idxg0)
        ff0, azr, px = _flow_stage(
            r1, px1, flow_b1, flow_g1, flow_be1,
            flow_w2, flow_b2, flow_g2, flow_be2,
            c0, st, p0, convz_w, convr_w)

        gzr = _sc_gather(azr, idxg0)
        sq, z, pxq = _gru_mid(gzr, px, c0, ff0, st, p0, convq_w,
                              convz_b, convr_b)
        gq = _sc_gather(sq, idxg0)
        gv, qq = _gru_fin(gq, pxq, z, st, p0, p1, f0, f1,
                          convq_b, interp_w1, interp_b1)

        r2 = _sc_gather(gv, idxg1)
        outs.append(_align(r2, qq, interp_w2, interp_b2))

    return jnp.stack(outs).transpose(0, 2, 1)


# speedup vs baseline: 1.0093x; 1.0073x over previous
import functools
import jax
import jax.numpy as jnp
from jax import lax
from jax.experimental import pallas as pl
from jax.experimental.pallas import tpu as pltpu
from jax.experimental.pallas import tpu_sc as plsc

NSAMPLE = 16
HID = 64
N = 4096
B = 2
QBLK = 256
EPS = 1e-5

_PAR1 = pltpu.CompilerParams(dimension_semantics=("parallel",))


def _leaky(x):
    return jnp.where(x >= 0, x, 0.1 * x)


def _knn_body(q_ref, rt_ref, o_ref):
    q = q_ref[...]
    rt = rt_ref[...]
    n = rt.shape[1]
    qn = jnp.sum(q * q, axis=1, keepdims=True)
    rn = jnp.sum(rt * rt, axis=0, keepdims=True)
    d = qn + rn - 2.0 * jnp.dot(q, rt, preferred_element_type=jnp.float32)
    iota = lax.broadcasted_iota(jnp.int32, d.shape, 1)
    cols = []
    for _ in range(NSAMPLE):
        m = jnp.min(d, axis=1, keepdims=True)
        mi = jnp.where(d <= m, iota, jnp.int32(n))
        j = jnp.min(mi, axis=1, keepdims=True)
        cols.append(j)
        d = jnp.where(iota == j, jnp.float32(jnp.inf), d)
    o_ref[...] = jnp.concatenate(cols, axis=1)


def _knn(queries, refs):
    idx = pl.pallas_call(
        _knn_body,
        grid=(N // QBLK,),
        in_specs=[pl.BlockSpec((QBLK, 3), lambda i: (i, 0)),
                  pl.BlockSpec((3, N), lambda i: (0, 0))],
        out_specs=pl.BlockSpec((QBLK, NSAMPLE), lambda i: (i, 0)),
        out_shape=jax.ShapeDtypeStruct((N, NSAMPLE), jnp.int32),
        compiler_params=_PAR1,
    )(queries, refs.T)
    return idx.reshape(-1)


def _sc_gather(table, idx):
    M = idx.shape[0]
    D = table.shape[1]
    NW = 32
    per_w = M // NW
    ch = min(per_w, 512)
    mesh = plsc.VectorSubcoreMesh(core_axis_name="c", subcore_axis_name="s")

    @functools.partial(
        pl.kernel, mesh=mesh,
        out_type=jax.ShapeDtypeStruct((M, D), jnp.float32),
        scratch_types=[
            pltpu.VMEM((ch,), jnp.int32),
            pltpu.VMEM((ch, D), jnp.float32),
            pltpu.SemaphoreType.DMA,
        ],
    )
    def k(table_hbm, idx_hbm, out_hbm, idx_v, rows_v, sem):
        wid = lax.axis_index("s") * 2 + lax.axis_index("c")
        base = wid * per_w

        @pl.loop(0, per_w, step=ch)
        def _(off):
            pltpu.sync_copy(idx_hbm.at[pl.ds(base + off, ch)], idx_v)
            pltpu.async_copy(table_hbm.at[idx_v], rows_v, sem).wait()
            pltpu.sync_copy(rows_v, out_hbm.at[pl.ds(base + off, ch)])

    return k(table, idx)


FBLK = 1024
FNB = N // FBLK
_FCNT = float(N * NSAMPLE * 16)


def _y1_block(r1, px, b1):
    pb = jnp.broadcast_to(px[:, None, :], (FBLK, NSAMPLE, 64)).reshape(
        FBLK * NSAMPLE, 64)
    return r1[:, 0:64] - pb + b1


def _gstats(y):
    parts = []
    for g in range(4):
        blkg = y[:, 16 * g:16 * (g + 1)]
        parts.append(jnp.sum(blkg).reshape(1, 1, 1))
    for g in range(4):
        blkg = y[:, 16 * g:16 * (g + 1)]
        parts.append(jnp.sum(blkg * blkg).reshape(1, 1, 1))
    return jnp.concatenate(parts, axis=2)


def _gn_apply(y, stats, gamma, beta):
    s = jnp.sum(stats.reshape(FNB, 8), axis=0)
    outs = []
    for g in range(4):
        m = s[g] / _FCNT
        v = s[4 + g] / _FCNT - m * m
        blkg = y[:, 16 * g:16 * (g + 1)]
        outs.append((blkg - m) * lax.rsqrt(v + EPS))
    yn = jnp.concatenate(outs, axis=1) * gamma + beta
    return _leaky(yn)


def _flow_s1_body(r1_ref, px_ref, b1_ref, st_ref):
    st_ref[...] = _gstats(_y1_block(r1_ref[...], px_ref[...], b1_ref[...]))


def _flow_s2_body(r1_ref, px_ref, b1_ref, st1_ref, g1_ref, be1_ref,
                  w2_ref, b2_ref, y2_ref, st2_ref):
    y1 = _y1_block(r1_ref[...], px_ref[...], b1_ref[...])
    h = _gn_apply(y1, st1_ref[...], g1_ref[...], be1_ref[...])
    y2 = jnp.dot(h, w2_ref[...].T, preferred_element_type=jnp.float32) \
        + b2_ref[...]
    y2_ref[...] = y2
    st2_ref[...] = _gstats(y2)


def _flow_s3_body(y2_ref, st2_ref, g2_ref, be2_ref, c_ref, s_ref, p_ref,
                  wz_ref, wr_ref, o_ref, azr_ref, px_ref):
    h2 = _gn_apply(y2_ref[...], st2_ref[...], g2_ref[...], be2_ref[...])
    ff = jnp.max(h2.reshape(FBLK, NSAMPLE, 64), axis=1)
    o_ref[...] = ff
    hs = jnp.concatenate([c_ref[...], ff, s_ref[...]], axis=1)
    p0 = p_ref[...]
    wz = wz_ref[...]
    wr = wr_ref[...]
    az = (jnp.dot(hs, wz[:, 0:192].T, preferred_element_type=jnp.float32)
          + jnp.dot(p0, wz[:, 192:195].T, preferred_element_type=jnp.float32))
    ar = (jnp.dot(hs, wr[:, 0:192].T, preferred_element_type=jnp.float32)
          + jnp.dot(p0, wr[:, 192:195].T, preferred_element_type=jnp.float32))
    azr_ref[...] = jnp.concatenate([az, ar], axis=1)
    pxz = jnp.dot(p0, wz[:, 192:195].T, preferred_element_type=jnp.float32)
    pxr = jnp.dot(p0, wr[:, 192:195].T, preferred_element_type=jnp.float32)
    px_ref[...] = jnp.concatenate([pxz, pxr], axis=1)


def _flow_stage(r1, px1, b1, g1, be1, w2, b2, g2, be2, c, s, p, wz, wr):
    grid = (FNB,)
    rblk = pl.BlockSpec((FBLK * NSAMPLE, 128), lambda i: (i, 0))
    yblk = pl.BlockSpec((FBLK * NSAMPLE, 64), lambda i: (i, 0))
    pblk = pl.BlockSpec((FBLK, 64), lambda i: (i, 0))
    vec = pl.BlockSpec((64,), lambda i: (0,))
    st_out = pl.BlockSpec((1, 1, 8), lambda i: (i, 0, 0))
    st_in = pl.BlockSpec((FNB, 1, 8), lambda i: (0, 0, 0))
    st_shape = jax.ShapeDtypeStruct((FNB, 1, 8), jnp.float32)

    st1 = pl.pallas_call(
        _flow_s1_body, grid=grid,
        in_specs=[rblk, pblk, vec],
        out_specs=st_out, out_shape=st_shape,
        compiler_params=_PAR1,
    )(r1, px1, b1)

    y2, st2 = pl.pallas_call(
        _flow_s2_body, grid=grid,
        in_specs=[rblk, pblk, vec, st_in, vec, vec,
                  pl.BlockSpec((64, 64), lambda i: (0, 0)), vec],
        out_specs=[yblk, st_out],
        out_shape=[jax.ShapeDtypeStruct((N * NSAMPLE, 64), jnp.float32),
                   st_shape],
        compiler_params=_PAR1,
    )(r1, px1, b1, st1, g1, be1, w2, b2)

    return pl.pallas_call(
        _flow_s3_body, grid=grid,
        in_specs=[yblk, st_in, vec, vec, pblk, pblk,
                  pl.BlockSpec((FBLK, 3), lambda i: (i, 0)),
                  pl.BlockSpec((64, 195), lambda i: (0, 0)),
                  pl.BlockSpec((64, 195), lambda i: (0, 0))],
        out_specs=[pblk, pl.BlockSpec((FBLK, 128), lambda i: (i, 0)),
                   pl.BlockSpec((FBLK, 128), lambda i: (i, 0))],
        out_shape=[jax.ShapeDtypeStruct((N, 64), jnp.float32),
                   jax.ShapeDtypeStruct((N, 128), jnp.float32),
                   jax.ShapeDtypeStruct((N, 128), jnp.float32)],
        compiler_params=_PAR1,
    )(y2, st2, g2, be2, c, s, p, wz, wr)


def _a1_body(fl_ref, p0_ref, w1_ref, a1_ref, px_ref):
    w1 = w1_ref[...]
    a1 = (jnp.dot(fl_ref[...], w1[:, 0:3].T, preferred_element_type=jnp.float32)
          + jnp.dot(p0_ref[...], w1[:, 3:6].T, preferred_element_type=jnp.float32))
    a1_ref[...] = jnp.concatenate(
        [a1, jnp.zeros((a1.shape[0], 64), jnp.float32)], axis=1)
    px_ref[...] = jnp.dot(p0_ref[...], w1[:, 3:6].T,
                          preferred_element_type=jnp.float32)


def _a1_pre(fl0, p0, w1):
    blk = lambda d: pl.BlockSpec((N, d), lambda: (0, 0))
    return pl.pallas_call(
        _a1_body,
        in_specs=[blk(3), blk(3), pl.BlockSpec((64, 6), lambda: (0, 0))],
        out_specs=[blk(128), blk(64)],
        out_shape=[jax.ShapeDtypeStruct((N, 128), jnp.float32),
                   jax.ShapeDtypeStruct((N, 64), jnp.float32)],
    )(fl0, p0, w1)


ZBLK = 1024
ZNB = N // ZBLK


def _gru_mid_body(g_ref, px_ref, c_ref, f_ref, s_ref, p_ref,
                  wq_ref, bz_ref, br_ref, sq_ref, z_ref, pxq_ref):
    mzr = jnp.max(g_ref[...].reshape(ZBLK, NSAMPLE, 128), axis=1)
    px = px_ref[...]
    z = jax.nn.sigmoid(mzr[:, 0:64] - px[:, 0:64] + bz_ref[...])
    r = jax.nn.sigmoid(mzr[:, 64:128] - px[:, 64:128] + br_ref[...])
    st = s_ref[...]
    rs = r * st
    feat = jnp.concatenate([c_ref[...], f_ref[...]], axis=1)
    wq = wq_ref[...]
    p0 = p_ref[...]
    sq = (jnp.dot(feat, wq[:, 0:128].T, preferred_element_type=jnp.float32)
          + jnp.dot(rs, wq[:, 128:192].T, preferred_element_type=jnp.float32)
          + jnp.dot(p0, wq[:, 192:195].T, preferred_element_type=jnp.float32))
    sq_ref[...] = jnp.concatenate(
        [sq, jnp.zeros((sq.shape[0], 64), jnp.float32)], axis=1)
    z_ref[...] = z
    pxq_ref[...] = jnp.dot(p0, wq[:, 192:195].T,
                           preferred_element_type=jnp.float32)


def _gru_mid(gzr, px, c, f, s, p, wq, bz, br):
    blk = lambda d: pl.BlockSpec((ZBLK, d), lambda g: (g, 0))
    vec = pl.BlockSpec((64,), lambda g: (0,))
    return pl.pallas_call(
        _gru_mid_body,
        grid=(ZNB,),
        in_specs=[pl.BlockSpec((ZBLK * NSAMPLE, 128), lambda g: (g, 0)),
                  blk(128), blk(64), blk(64), blk(64), blk(3),
                  pl.BlockSpec((64, 195), lambda g: (0, 0)), vec, vec],
        out_specs=[blk(128), blk(64), blk(64)],
        out_shape=[jax.ShapeDtypeStruct((N, 128), jnp.float32),
                   jax.ShapeDtypeStruct((N, 64), jnp.float32),
                   jax.ShapeDtypeStruct((N, 64), jnp.float32)],
        compiler_params=_PAR1,
    )(gzr, px, c, f, s, p, wq, bz, br)


def _gru_fin_body(g_ref, pxq_ref, z_ref, s_ref, p0_ref, p1_ref,
                  f0_ref, f1_ref, bq_ref, wa_ref, ba_ref, gv_ref, qq_ref):
    mq = jnp.max(g_ref[...][:, 0:64].reshape(ZBLK, NSAMPLE, 64), axis=1)
    q = jnp.tanh(mq - pxq_ref[...] + bq_ref[...])
    z = z_ref[...]
    ns = (1.0 - z) * s_ref[...] + z * q
    wa = wa_ref[...]
    g_t = (jnp.dot(f0_ref[...], wa[:, 0:64].T, preferred_element_type=jnp.float32)
           + jnp.dot(p0_ref[...], wa[:, 128:131].T, preferred_element_type=jnp.float32))
    qq = (jnp.dot(f1_ref[...], wa[:, 64:128].T, preferred_element_type=jnp.float32)
          - jnp.dot(p1_ref[...], wa[:, 128:131].T, preferred_element_type=jnp.float32)
          + ba_ref[...])
    gv_ref[...] = jnp.concatenate([g_t, ns], axis=1)
    qq_ref[...] = qq


def _gru_fin(gq, pxq, z, s, p0, p1, f0, f1, bq, wa, ba):
    blk = lambda d: pl.BlockSpec((ZBLK, d), lambda g: (g, 0))
    vec = pl.BlockSpec((64,), lambda g: (0,))
    return pl.pallas_call(
        _gru_fin_body,
        grid=(ZNB,),
        in_specs=[pl.BlockSpec((ZBLK * NSAMPLE, 128), lambda g: (g, 0)),
                  blk(64), blk(64), blk(64), blk(3), blk(3), blk(64), blk(64),
                  vec, pl.BlockSpec((64, 131), lambda g: (0, 0)), vec],
        out_specs=[blk(128), blk(64)],
        out_shape=[jax.ShapeDtypeStruct((N, 128), jnp.float32),
                   jax.ShapeDtypeStruct((N, 64), jnp.float32)],
        compiler_params=_PAR1,
    )(gq, pxq, z, s, p0, p1, f0, f1, bq, wa, ba)


ABLK = 512


def _align_body(r2_ref, qq_ref, w2_ref, b2_ref, o_ref):
    r2 = r2_ref[...]
    qq = qq_ref[...]
    qb = jnp.broadcast_to(qq[:, None, :], (ABLK, NSAMPLE, 64)).reshape(
        ABLK * NSAMPLE, 64)
    h = _leaky(r2[:, 0:64] + qb)
    y = jnp.dot(h, w2_ref[...].T, preferred_element_type=jnp.float32) + b2_ref[...]
    y3 = y.reshape(ABLK, NSAMPLE, 64)
    m = jnp.max(y3, axis=1, keepdims=True)
    e = jnp.exp(y3 - m)
    w = e / jnp.sum(e, axis=1, keepdims=True)
    v3 = r2[:, 64:128].reshape(ABLK, NSAMPLE, 64)
    o_ref[...] = jnp.sum(w * v3, axis=1)


def _align(r2, qq, w2, b2):
    return pl.pallas_call(
        _align_body,
        grid=(N // ABLK,),
        in_specs=[pl.BlockSpec((ABLK * NSAMPLE, 128), lambda g: (g, 0)),
                  pl.BlockSpec((ABLK, 64), lambda g: (g, 0)),
                  pl.BlockSpec((64, 64), lambda g: (0, 0)),
                  pl.BlockSpec((64,), lambda g: (0,))],
        out_specs=pl.BlockSpec((ABLK, 64), lambda g: (g, 0)),
        out_shape=jax.ShapeDtypeStruct((N, 64), jnp.float32),
        compiler_params=_PAR1,
    )(r2, qq, w2, b2)


def kernel(xyz0, xyz1, state, corr0, feat0, feat1, flow0,
           flow_w1, flow_b1, flow_g1, flow_be1,
           flow_w2, flow_b2, flow_g2, flow_be2,
           convz_w, convz_b, convr_w, convr_b, convq_w, convq_b,
           interp_w1, interp_b1, interp_w2, interp_b2):
    t = lambda x: x.transpose(0, 2, 1)
    p0a = t(xyz0)
    p1a = t(xyz1)
    sta = t(state)
    c0a = t(corr0)
    f0a = t(feat0)
    f1a = t(feat1)
    fla = t(flow0)

    outs = []
    for b in range(B):
        p0, p1, st, c0 = p0a[b], p1a[b], sta[b], c0a[b]
        f0, f1, fl = f0a[b], f1a[b], fla[b]

        idxg0 = _knn(p0, p0)
        idxg1 = _knn(p1, p0)

        a1, px1 = _a1_pre(fl, p0, flow_w1)
        r1 = _sc_gather(a1, idxg0)
        ff0, azr, px = _flow_stage(
            r1, px1, flow_b1, flow_g1, flow_be1,
            flow_w2, flow_b2, flow_g2, flow_be2,
            c0, st, p0, convz_w, convr_w)

        gzr = _sc_gather(azr, idxg0)
        sq, z, pxq = _gru_mid(gzr, px, c0, ff0, st, p0, convq_w,
                              convz_b, convr_b)
        gq = _sc_gather(sq, idxg0)
        gv, qq = _gru_fin(gq, pxq, z, st, p0, p1, f0, f1,
                          convq_b, interp_w1, interp_b1)

        r2 = _sc_gather(gv, idxg1)
        outs.append(_align(r2, qq, interp_w2, interp_b2))

    return jnp.stack(outs).transpose(0, 2, 1)
